# baseline (speedup 1.0000x reference)
import jax
import jax.numpy as jnp
from jax import lax
from jax.experimental import pallas as pl
from jax.experimental.pallas import tpu as pltpu
from jax.experimental.pallas import tpu_sc as plsc

_EMB_DIM = 64
_NUM_WORKERS = 32
_CHUNK = 256
_DEPTH = 2


def _make_gather(batch):
    bpw = batch // _NUM_WORKERS
    nchunk = bpw // _CHUNK
    nbuf = 2 * _DEPTH
    mesh = plsc.VectorSubcoreMesh(core_axis_name="c", subcore_axis_name="s")

    def body(idx_hbm, table_hbm, out_hbm, idx_v, rows_v, *sems):
        gsem = sems[:nbuf]
        wsem = sems[nbuf:]
        wid = lax.axis_index("s") * 2 + lax.axis_index("c")
        base = wid * bpw
        pltpu.sync_copy(idx_hbm.at[pl.ds(base, bpw)], idx_v)

        def gather(j, b):
            idx_slice = idx_v.at[pl.ds(j * _CHUNK, _CHUNK)]
            return pltpu.make_async_copy(
                table_hbm.at[idx_slice], rows_v.at[b], gsem[b])

        def write(j, b):
            return pltpu.make_async_copy(
                rows_v.at[b], out_hbm.at[pl.ds(base + j * _CHUNK, _CHUNK)],
                wsem[b])

        for b in range(_DEPTH):
            gather(b, b).start()

        def outer(i, carry):
            for k in range(nbuf):
                j = i * nbuf + k
                gather(j, k).wait()
                write(j, k).start()
                b2 = (k + _DEPTH) % nbuf
                jn = j + _DEPTH

                @pl.when(j >= _DEPTH)
                def _():
                    write(j - _DEPTH, b2).wait()

                @pl.when(jn < nchunk)
                def _():
                    gather(jn, b2).start()
            return carry

        lax.fori_loop(0, nchunk // nbuf, outer, 0)

        for k in range(_DEPTH):
            j = nchunk - _DEPTH + k
            write(j, j % nbuf).wait()

    return pl.kernel(
        body,
        out_type=jax.ShapeDtypeStruct((batch, _EMB_DIM), jnp.float32),
        mesh=mesh,
        compiler_params=pltpu.CompilerParams(use_tc_tiling_on_sc=False),
        scratch_types=(
            [
                pltpu.VMEM((bpw,), jnp.int32),
                pltpu.VMEM((nbuf, _CHUNK, _EMB_DIM), jnp.float32),
            ]
            + [pltpu.SemaphoreType.DMA] * (2 * nbuf)
        ),
    )


def kernel(token_ids, emb_matrix):
    seq, toks = token_ids.shape
    batch = seq * toks
    flat = token_ids.reshape(batch).astype(jnp.int32)
    out = _make_gather(batch)(flat, emb_matrix)
    return out.reshape(seq, toks, _EMB_DIM)
